# baseline (speedup 1.0000x reference)
import functools

import jax
import jax.numpy as jnp
from jax import lax
from jax.experimental import pallas as pl
from jax.experimental.pallas import tpu as pltpu
from jax.experimental.pallas import tpu_sc as plsc

_NC = 2
_NS = 16
_L = 16
_NW = _NC * _NS

_B = 16384
_NNUM = 13
_NCAT = 26
_CATDIM = 1000
_D = 128
_NSLOT = 1 + _NNUM + _NCAT
_NDEN = 1 + _NNUM

_BPW = _B // _NW
_CB = 4
_NCHUNK = _BPW // _CB
_NBUF = 4

_QS = _D // _L
_CG = _CB * _NCAT
_CN = _CB * _NDEN

_mesh = plsc.VectorSubcoreMesh(core_axis_name="c", subcore_axis_name="s")


@functools.partial(
    pl.kernel,
    mesh=_mesh,
    compiler_params=pltpu.CompilerParams(needs_layout_passes=False),
    out_type=jax.ShapeDtypeStruct((_B * _NSLOT, _D), jnp.float32),
    scratch_types=[
        pltpu.VMEM((_BPW * _NCAT,), jnp.int32),
        pltpu.VMEM((_BPW * _NCAT,), jnp.int32),
        pltpu.VMEM((_BPW * _NDEN,), jnp.int32),
        pltpu.VMEM((_NBUF, _CG, _D), jnp.float32),
        pltpu.VMEM((_NBUF, _CN, _D), jnp.float32),
        pltpu.VMEM((_BPW * _NNUM,), jnp.float32),
        pltpu.VMEM((_NNUM, _D), jnp.float32),
        pltpu.VMEM((_NNUM, _D), jnp.float32),
        pltpu.VMEM((1, _D), jnp.float32),
        pltpu.SemaphoreType.DMA,
        pltpu.SemaphoreType.DMA,
        pltpu.SemaphoreType.DMA,
    ],
)
def _tokenize_sc(xnum_hbm, idx_hbm, oidx_hbm, nidx_hbm, w_hbm, bn_hbm,
                 tab_hbm, cls_hbm, out_hbm,
                 idx_v, oidx_v, nidx_v, cat_buf, num_buf, xn_v, w_v, bn_v,
                 cls_v, gsem, osem, ssem):
    wid = lax.axis_index("s") * _NC + lax.axis_index("c")
    base0 = wid * _BPW

    def issue_gather(c, par):
        pltpu.async_copy(
            tab_hbm.at[idx_v.at[pl.ds(c * _CG, _CG)]],
            cat_buf.at[par], gsem)

    def wait_gather(par):
        pltpu.make_async_copy(
            tab_hbm.at[idx_v.at[pl.ds(0, _CG)]], cat_buf.at[par],
            gsem).wait()

    def issue_scatters(c, par):
        pltpu.async_copy(cat_buf.at[par],
                         out_hbm.at[oidx_v.at[pl.ds(c * _CG, _CG)]], osem)
        pltpu.async_copy(num_buf.at[par],
                         out_hbm.at[nidx_v.at[pl.ds(c * _CN, _CN)]], osem)

    def wait_scatters(par):
        pltpu.make_async_copy(cat_buf.at[par],
                              out_hbm.at[oidx_v.at[pl.ds(0, _CG)]],
                              osem).wait()
        pltpu.make_async_copy(num_buf.at[par],
                              out_hbm.at[nidx_v.at[pl.ds(0, _CN)]],
                              osem).wait()

    pltpu.async_copy(idx_hbm.at[pl.ds(base0 * _NCAT, _BPW * _NCAT)], idx_v,
                     ssem)
    pltpu.async_copy(oidx_hbm.at[pl.ds(base0 * _NCAT, _BPW * _NCAT)], oidx_v,
                     ssem)
    pltpu.async_copy(nidx_hbm.at[pl.ds(base0 * _NDEN, _BPW * _NDEN)], nidx_v,
                     ssem)
    pltpu.async_copy(xnum_hbm.at[pl.ds(base0 * _NNUM, _BPW * _NNUM)], xn_v,
                     ssem)
    pltpu.async_copy(w_hbm, w_v, ssem)
    pltpu.async_copy(bn_hbm, bn_v, ssem)
    pltpu.async_copy(cls_hbm, cls_v, ssem)
    pltpu.make_async_copy(idx_hbm.at[pl.ds(0, _BPW * _NCAT)], idx_v,
                          ssem).wait()
    pltpu.make_async_copy(oidx_hbm.at[pl.ds(0, _BPW * _NCAT)], oidx_v,
                          ssem).wait()
    pltpu.make_async_copy(nidx_hbm.at[pl.ds(0, _BPW * _NDEN)], nidx_v,
                          ssem).wait()
    pltpu.make_async_copy(xnum_hbm.at[pl.ds(0, _BPW * _NNUM)], xn_v,
                          ssem).wait()
    pltpu.make_async_copy(w_hbm, w_v, ssem).wait()
    pltpu.make_async_copy(bn_hbm, bn_v, ssem).wait()
    pltpu.make_async_copy(cls_hbm, cls_v, ssem).wait()

    for q in range(_QS):
        cv = cls_v[0, pl.ds(q * _L, _L)]
        for p in range(_NBUF):
            for b in range(_CB):
                num_buf[p, b * _NDEN, pl.ds(q * _L, _L)] = cv

    issue_gather(0, 0)

    def chunk_body(c, _):
        par = lax.rem(c, _NBUF)
        nxt = lax.rem(c + 1, _NBUF)

        @pl.when(c + 1 < _NCHUNK)
        def _():
            @pl.when(c + 1 >= _NBUF)
            def _():
                wait_scatters(nxt)

            issue_gather(c + 1, nxt)

        for j in range(_NNUM):
            wq = [w_v[j, pl.ds(q * _L, _L)] for q in range(_QS)]
            bq = [bn_v[j, pl.ds(q * _L, _L)] for q in range(_QS)]

            def num_body(b, _, j=j, wq=wq, bq=bq):
                fi = jnp.full((_L,), (c * _CB + b) * _NNUM + j, jnp.int32)
                xs = plsc.load_gather(xn_v, [fi])
                for q in range(_QS):
                    num_buf[par, b * _NDEN + 1 + j, pl.ds(q * _L, _L)] = (
                        xs * wq[q] + bq[q])
                return 0

            lax.fori_loop(0, _CB, num_body, 0)

        wait_gather(par)
        issue_scatters(c, par)
        return 0

    lax.fori_loop(0, _NCHUNK, chunk_body, 0)

    for p in range(_NBUF):
        wait_scatters(p)


def kernel(x_num, x_cat, num_weights, num_biases, cat_tables, cat_biases,
           cls_token):
    offs = (jnp.arange(_NCAT, dtype=jnp.int32) * _CATDIM)[None, :]
    idx = (x_cat.astype(jnp.int32) + offs).reshape(-1)
    brow = jnp.arange(_B, dtype=jnp.int32)[:, None] * _NSLOT
    oidx = (brow + 1 + _NNUM
            + jnp.arange(_NCAT, dtype=jnp.int32)[None, :]).reshape(-1)
    nidx = (brow + jnp.arange(_NDEN, dtype=jnp.int32)[None, :]).reshape(-1)
    tab = (cat_tables + cat_biases[:, None, :]).reshape(_NCAT * _CATDIM, _D)
    cls = cls_token.reshape(1, _D)
    out = _tokenize_sc(x_num.reshape(-1), idx, oidx, nidx, num_weights,
                       num_biases, tab, cls)
    return out.reshape(_B, _NSLOT, _D)


# speedup vs baseline: 1.1160x; 1.1160x over previous
import functools

import jax
import jax.numpy as jnp
from jax import lax
from jax.experimental import pallas as pl
from jax.experimental.pallas import tpu as pltpu
from jax.experimental.pallas import tpu_sc as plsc

_NC = 2
_NS = 16
_L = 16
_NW = _NC * _NS

_B = 16384
_NNUM = 13
_NCAT = 26
_CATDIM = 1000
_D = 128
_NSLOT = 1 + _NNUM + _NCAT

_BPW = _B // _NW
_CB = 4
_NCHUNK = _BPW // _CB
_NBUF = 4

_QS = _D // _L
_IP = 32
_XP = 16

_mesh = plsc.VectorSubcoreMesh(core_axis_name="c", subcore_axis_name="s")


@functools.partial(
    pl.kernel,
    mesh=_mesh,
    compiler_params=pltpu.CompilerParams(needs_layout_passes=False),
    out_type=jax.ShapeDtypeStruct((_B, _NSLOT, _D), jnp.float32),
    scratch_types=[
        pltpu.VMEM((_BPW * _IP,), jnp.int32),
        pltpu.VMEM((_NBUF, _CB, _NSLOT, _D), jnp.float32),
        pltpu.VMEM((_BPW * _XP,), jnp.float32),
        pltpu.VMEM((_NNUM, _D), jnp.float32),
        pltpu.VMEM((_NNUM, _D), jnp.float32),
        pltpu.VMEM((1, _D), jnp.float32),
        pltpu.SemaphoreType.DMA,
        pltpu.SemaphoreType.DMA,
        pltpu.SemaphoreType.DMA,
    ],
)
def _tokenize_sc(xnum_hbm, idx_hbm, w_hbm, bn_hbm, tab_hbm, cls_hbm, out_hbm,
                 idx_v, stage_buf, xn_v, w_v, bn_v, cls_v,
                 gsem, osem, ssem):
    wid = lax.axis_index("s") * _NC + lax.axis_index("c")
    base0 = wid * _BPW

    def issue_gathers(c, par):
        for b in range(_CB):
            pltpu.async_copy(
                tab_hbm.at[idx_v.at[pl.ds((c * _CB + b) * _IP, _NCAT)]],
                stage_buf.at[par, b, pl.ds(1 + _NNUM, _NCAT)], gsem)

    def wait_gathers(par):
        for b in range(_CB):
            pltpu.make_async_copy(
                tab_hbm.at[idx_v.at[pl.ds(b * _IP, _NCAT)]],
                stage_buf.at[par, b, pl.ds(1 + _NNUM, _NCAT)], gsem).wait()

    def wait_out(par):
        pltpu.make_async_copy(stage_buf.at[par], out_hbm.at[pl.ds(0, _CB)],
                              osem).wait()

    pltpu.async_copy(idx_hbm.at[pl.ds(base0 * _IP, _BPW * _IP)], idx_v, ssem)
    pltpu.async_copy(xnum_hbm.at[pl.ds(base0 * _XP, _BPW * _XP)], xn_v,
                     ssem)
    pltpu.async_copy(w_hbm, w_v, ssem)
    pltpu.async_copy(bn_hbm, bn_v, ssem)
    pltpu.async_copy(cls_hbm, cls_v, ssem)
    pltpu.make_async_copy(idx_hbm.at[pl.ds(0, _BPW * _IP)], idx_v,
                          ssem).wait()
    pltpu.make_async_copy(xnum_hbm.at[pl.ds(0, _BPW * _XP)], xn_v,
                          ssem).wait()
    pltpu.make_async_copy(w_hbm, w_v, ssem).wait()
    pltpu.make_async_copy(bn_hbm, bn_v, ssem).wait()
    pltpu.make_async_copy(cls_hbm, cls_v, ssem).wait()

    for q in range(_QS):
        cv = cls_v[0, pl.ds(q * _L, _L)]
        for p in range(_NBUF):

            def fill_b(b, _, cv=cv, q=q, p=p):
                stage_buf[p, b, 0, pl.ds(q * _L, _L)] = cv
                return 0

            lax.fori_loop(0, _CB, fill_b, 0)

    issue_gathers(0, 0)

    def chunk_body(c, _):
        par = lax.rem(c, _NBUF)
        nxt = lax.rem(c + 1, _NBUF)
        base = base0 + c * _CB

        @pl.when(c + 1 < _NCHUNK)
        def _():
            @pl.when(c + 1 >= _NBUF)
            def _():
                wait_out(nxt)

            issue_gathers(c + 1, nxt)

        xc = xn_v.at[pl.ds(c * _CB * _XP, _CB * _XP)]
        for j in range(_NNUM):
            wq = [w_v[j, pl.ds(q * _L, _L)] for q in range(_QS)]
            bq = [bn_v[j, pl.ds(q * _L, _L)] for q in range(_QS)]
            xs = [
                plsc.load_gather(
                    xc, [jnp.full((_L,), b * _XP + j, jnp.int32)])
                for b in range(_CB)
            ]
            for b in range(_CB):
                for q in range(_QS):
                    stage_buf[par, b, 1 + j, pl.ds(q * _L, _L)] = (
                        xs[b] * wq[q] + bq[q])

        wait_gathers(par)

        pltpu.async_copy(stage_buf.at[par], out_hbm.at[pl.ds(base, _CB)], osem)
        return 0

    lax.fori_loop(0, _NCHUNK, chunk_body, 0)

    for p in range(_NBUF):
        wait_out(p)


def kernel(x_num, x_cat, num_weights, num_biases, cat_tables, cat_biases,
           cls_token):
    offs = (jnp.arange(_NCAT, dtype=jnp.int32) * _CATDIM)[None, :]
    idx = x_cat.astype(jnp.int32) + offs
    idx = jnp.pad(idx, ((0, 0), (0, _IP - _NCAT))).reshape(-1)
    xn = jnp.pad(x_num, ((0, 0), (0, _XP - _NNUM))).reshape(-1)
    tab = (cat_tables + cat_biases[:, None, :]).reshape(_NCAT * _CATDIM, _D)
    cls = cls_token.reshape(1, _D)
    return _tokenize_sc(xn, idx, num_weights, num_biases, tab, cls)


# speedup vs baseline: 1.1369x; 1.0188x over previous
import functools

import jax
import jax.numpy as jnp
from jax import lax
from jax.experimental import pallas as pl
from jax.experimental.pallas import tpu as pltpu
from jax.experimental.pallas import tpu_sc as plsc

_NC = 2
_NS = 16
_L = 16
_NW = _NC * _NS

_B = 16384
_NNUM = 13
_NCAT = 26
_CATDIM = 1000
_D = 128
_NSLOT = 1 + _NNUM + _NCAT

_BPW = _B // _NW
_CB = 4
_NCHUNK = _BPW // _CB
_NBUF = 5

_QS = _D // _L
_IP = 32

_mesh = plsc.VectorSubcoreMesh(core_axis_name="c", subcore_axis_name="s")


@functools.partial(
    pl.kernel,
    mesh=_mesh,
    compiler_params=pltpu.CompilerParams(needs_layout_passes=False),
    out_type=jax.ShapeDtypeStruct((_B, _NSLOT, _D), jnp.float32),
    scratch_types=[
        pltpu.VMEM((_BPW * _IP,), jnp.int32),
        pltpu.VMEM((_NBUF, _CB, _NSLOT, _D), jnp.float32),
        pltpu.VMEM((_BPW * _NNUM,), jnp.float32),
        pltpu.VMEM((_NNUM, _D), jnp.float32),
        pltpu.VMEM((_NNUM, _D), jnp.float32),
        pltpu.VMEM((1, _D), jnp.float32),
        pltpu.SemaphoreType.DMA,
        pltpu.SemaphoreType.DMA,
        pltpu.SemaphoreType.DMA,
    ],
)
def _tokenize_sc(xnum_hbm, idx_hbm, w_hbm, bn_hbm, tab_hbm, cls_hbm, out_hbm,
                 idx_v, stage_buf, xn_v, w_v, bn_v, cls_v,
                 gsem, osem, ssem):
    wid = lax.axis_index("s") * _NC + lax.axis_index("c")
    base0 = wid * _BPW

    def issue_gathers(c, par):
        for b in range(_CB):
            pltpu.async_copy(
                tab_hbm.at[idx_v.at[pl.ds((c * _CB + b) * _IP, _NCAT)]],
                stage_buf.at[par, b, pl.ds(1 + _NNUM, _NCAT)], gsem)

    def wait_gathers(par):
        for b in range(_CB):
            pltpu.make_async_copy(
                tab_hbm.at[idx_v.at[pl.ds(b * _IP, _NCAT)]],
                stage_buf.at[par, b, pl.ds(1 + _NNUM, _NCAT)], gsem).wait()

    def wait_out(par):
        pltpu.make_async_copy(stage_buf.at[par], out_hbm.at[pl.ds(0, _CB)],
                              osem).wait()

    pltpu.async_copy(idx_hbm.at[pl.ds(base0 * _IP, _BPW * _IP)], idx_v, ssem)
    pltpu.async_copy(xnum_hbm.at[pl.ds(base0 * _NNUM, _BPW * _NNUM)], xn_v,
                     ssem)
    pltpu.async_copy(w_hbm, w_v, ssem)
    pltpu.async_copy(bn_hbm, bn_v, ssem)
    pltpu.async_copy(cls_hbm, cls_v, ssem)
    pltpu.make_async_copy(idx_hbm.at[pl.ds(0, _BPW * _IP)], idx_v,
                          ssem).wait()
    pltpu.make_async_copy(xnum_hbm.at[pl.ds(0, _BPW * _NNUM)], xn_v,
                          ssem).wait()
    pltpu.make_async_copy(w_hbm, w_v, ssem).wait()
    pltpu.make_async_copy(bn_hbm, bn_v, ssem).wait()
    pltpu.make_async_copy(cls_hbm, cls_v, ssem).wait()

    for q in range(_QS):
        cv = cls_v[0, pl.ds(q * _L, _L)]
        for p in range(_NBUF):

            def fill_b(b, _, cv=cv, q=q, p=p):
                stage_buf[p, b, 0, pl.ds(q * _L, _L)] = cv
                return 0

            lax.fori_loop(0, _CB, fill_b, 0)

    issue_gathers(0, 0)

    def chunk_body(c, _):
        par = lax.rem(c, _NBUF)
        nxt = lax.rem(c + 1, _NBUF)
        base = base0 + c * _CB

        @pl.when(c + 1 < _NCHUNK)
        def _():
            @pl.when(c + 1 >= _NBUF)
            def _():
                wait_out(nxt)

            issue_gathers(c + 1, nxt)

        for j in range(_NNUM):
            wq = [w_v[j, pl.ds(q * _L, _L)] for q in range(_QS)]
            bq = [bn_v[j, pl.ds(q * _L, _L)] for q in range(_QS)]

            def num_body(b, _, j=j, wq=wq, bq=bq):
                fi = jnp.full((_L,), (c * _CB + b) * _NNUM + j, jnp.int32)
                xs = plsc.load_gather(xn_v, [fi])
                for q in range(_QS):
                    stage_buf[par, b, 1 + j, pl.ds(q * _L, _L)] = (
                        xs * wq[q] + bq[q])
                return 0

            lax.fori_loop(0, _CB, num_body, 0)

        wait_gathers(par)

        pltpu.async_copy(stage_buf.at[par], out_hbm.at[pl.ds(base, _CB)], osem)
        return 0

    lax.fori_loop(0, _NCHUNK, chunk_body, 0)

    for p in range(_NBUF):
        wait_out(p)


def kernel(x_num, x_cat, num_weights, num_biases, cat_tables, cat_biases,
           cls_token):
    offs = (jnp.arange(_NCAT, dtype=jnp.int32) * _CATDIM)[None, :]
    idx = x_cat.astype(jnp.int32) + offs
    idx = jnp.pad(idx, ((0, 0), (0, _IP - _NCAT))).reshape(-1)
    tab = (cat_tables + cat_biases[:, None, :]).reshape(_NCAT * _CATDIM, _D)
    cls = cls_token.reshape(1, _D)
    return _tokenize_sc(x_num.reshape(-1), idx, num_weights, num_biases, tab,
                        cls)
